# Initial kernel scaffold; baseline (speedup 1.0000x reference)
import jax
import jax.numpy as jnp
from jax.experimental import pallas as pl


def kernel(x1, edge_index1, e1, u1, batch1, x2, edge_index2, e2, u2, batch2, params):
    raise NotImplementedError("write your pallas kernel here")


# speedup vs baseline: 3.3342x; 3.3342x over previous
import functools

import jax
import jax.numpy as jnp
from jax import lax
from jax.experimental import pallas as pl
from jax.experimental.pallas import tpu as pltpu
from jax.experimental.pallas import tpu_sc as plsc

N = 10000
E = 320000
B = 8
F_X = 128
F_E = 16
H = 128

NC = 2
NS = 16
NW = NC * NS

NPAD = 10240
EPAD = 327680
KG = 80
K = 128
RPC = NPAD // NC
SROWS = RPC + 128

_f32 = jnp.float32
_bf16 = jnp.bfloat16


@functools.cache
def _sc_mesh():
    return plsc.VectorSubcoreMesh(
        core_axis_name="c", subcore_axis_name="s", num_cores=NC,
        num_subcores=NS)


def _dot(a, w):
    return jnp.dot(a.astype(_bf16), w, preferred_element_type=_f32)


def _prep_body(x_ref, bc_ref, u_ref, w1s_ref, w1d_ref, w1u_ref, b1_ref,
               pa_ref, xb_ref):
    x = x_ref[...]
    oh = (lax.broadcasted_iota(jnp.int32, (x.shape[0], B), 1) == bc_ref[...]
          ).astype(_bf16)
    uw = _dot(u_ref[...], w1u_ref[...])
    pa_ref[...] = (_dot(x, w1s_ref[...]) +
                   jnp.dot(oh, uw.astype(_bf16), preferred_element_type=_f32) +
                   b1_ref[...])
    xb_ref[...] = _dot(x, w1d_ref[...])


def _prep_call(xp, bc, u, w1s, w1d, w1u, b1):
    blk = 1024
    grid = NPAD // blk
    return pl.pallas_call(
        _prep_body,
        grid=(grid,),
        in_specs=[
            pl.BlockSpec((blk, F_X), lambda i: (i, 0)),
            pl.BlockSpec((blk, 1), lambda i: (i, 0)),
            pl.BlockSpec((B, F_X), lambda i: (0, 0)),
            pl.BlockSpec((F_X, H), lambda i: (0, 0)),
            pl.BlockSpec((F_X, H), lambda i: (0, 0)),
            pl.BlockSpec((F_X, H), lambda i: (0, 0)),
            pl.BlockSpec((1, H), lambda i: (0, 0)),
        ],
        out_specs=[
            pl.BlockSpec((blk, H), lambda i: (i, 0)),
            pl.BlockSpec((blk, H), lambda i: (i, 0)),
        ],
        out_shape=[
            jax.ShapeDtypeStruct((NPAD, H), _f32),
            jax.ShapeDtypeStruct((NPAD, H), _f32),
        ],
    )(xp, bc, u, w1s, w1d, w1u, b1)


def _gather_sc(pa_all, xb_all, srcs, dsts):
    @functools.partial(
        pl.kernel,
        out_type=[
            jax.ShapeDtypeStruct((2 * EPAD, H), _f32),
            jax.ShapeDtypeStruct((2 * EPAD, H), _f32),
        ],
        mesh=_sc_mesh(),
        scratch_types=[],
    )
    def kern(pa_h, xb_h, src_h, dst_h, ga_h, gb_h):
        def body(ia, ib, oa, ob):
            pltpu.sync_copy(pa_h.at[ia.at[0]], oa)
            pltpu.sync_copy(xb_h.at[ib.at[0]], ob)

        pltpu.emit_pipeline(
            body,
            grid=(2 * EPAD // K,),
            in_specs=[
                pl.BlockSpec((1, K), lambda i: (0, i)),
                pl.BlockSpec((1, K), lambda i: (0, i)),
            ],
            out_specs=[
                pl.BlockSpec((K, H), lambda i: (i, 0)),
                pl.BlockSpec((K, H), lambda i: (i, 0)),
            ],
            core_axis_name=("c", "s"),
            dimension_semantics=(pltpu.PARALLEL,),
        )(src_h, dst_h, ga_h, gb_h)

    return kern(pa_all, xb_all, srcs, dsts)


def _edge_body(ga_ref, gb_ref, e_ref, w1e_ref, w2_ref, b2_ref, w3_ref, b3_ref,
               eh_ref):
    h1 = ga_ref[...] + gb_ref[...] + _dot(e_ref[...], w1e_ref[...])
    h1 = jnp.maximum(h1, 0.0)
    h2 = jnp.maximum(_dot(h1, w2_ref[...]) + b2_ref[...], 0.0)
    eh_ref[...] = _dot(h2, w3_ref[...]) + b3_ref[...]


def _edge_call(ga, gb, ep, w1e, w2, b2, w3, b3, goff):
    blk = 2048
    grid = EPAD // blk
    ob = goff // blk
    return pl.pallas_call(
        _edge_body,
        grid=(grid,),
        in_specs=[
            pl.BlockSpec((blk, H), lambda i: (i + ob, 0)),
            pl.BlockSpec((blk, H), lambda i: (i + ob, 0)),
            pl.BlockSpec((blk, F_E), lambda i: (i, 0)),
            pl.BlockSpec((F_E, H), lambda i: (0, 0)),
            pl.BlockSpec((H, H), lambda i: (0, 0)),
            pl.BlockSpec((1, H), lambda i: (0, 0)),
            pl.BlockSpec((H, H), lambda i: (0, 0)),
            pl.BlockSpec((1, H), lambda i: (0, 0)),
        ],
        out_specs=pl.BlockSpec((blk, H), lambda i: (i, 0)),
        out_shape=jax.ShapeDtypeStruct((EPAD, H), _f32),
    )(ga, gb, ep, w1e, w2, b2, w3, b3)


def _scatter_sc(dst2d_1, eh_1, dst2d_2, eh_2):
    SEPT = EPAD // NS
    CH = SEPT // K
    ZB = SROWS // NS // 8
    WROWS = RPC // NS

    @functools.partial(
        pl.kernel,
        out_type=[
            jax.ShapeDtypeStruct((NC, RPC, H), _f32),
            jax.ShapeDtypeStruct((NC, RPC, H), _f32),
            jax.ShapeDtypeStruct((NC, RPC, H), _f32),
            jax.ShapeDtypeStruct((NC, RPC, H), _f32),
        ],
        mesh=_sc_mesh(),
        scratch_types=[
            pltpu.VMEM((8, H), _f32),
            pltpu.VMEM((K, H), _f32),
            pltpu.VMEM((K,), jnp.int32),
            pltpu.VMEM((K,), jnp.int32),
            pltpu.VMEM_SHARED((SROWS, H), _f32),
        ],
    )
    def kern(dst1_hbm, eh1_hbm, dst2_hbm, eh2_hbm,
             oagg1_hbm, ocnt1_hbm, oagg2_hbm, ocnt2_hbm,
             zrow_v, eb0, ib0, lb0, sagg):
        cid = lax.axis_index("c")
        sid = lax.axis_index("s")
        base = sid * SEPT

        @pl.loop(0, 8)
        def _fill(r):
            @pl.loop(0, H // 16)
            def _z(cc):
                zrow_v[r, pl.ds(cc * 16, 16)] = jnp.zeros((16,), _f32)

        def fill_ones():
            @pl.loop(0, K)
            def _o1(r):
                @pl.loop(0, H // 16)
                def _o2(cc):
                    eb0[r, pl.ds(cc * 16, 16)] = jnp.full((16,), 1.0, _f32)

        zbase = sid * (SROWS // NS)

        def zero_rows():
            @pl.loop(0, ZB)
            def _zero(zi):
                pltpu.sync_copy(zrow_v, sagg.at[pl.ds(zbase + zi * 8, 8)])

        ebufs = (eb0,)
        ibufs = (ib0,)
        lbufs = (lb0,)

        def rewrite(b):
            @pl.loop(0, K // 16)
            def _rw(j):
                v = ibufs[b][pl.ds(j * 16, 16)]
                v = v - cid * RPC
                oob = (v < 0) | (v >= RPC)
                lbufs[b][pl.ds(j * 16, 16)] = jnp.where(oob, RPC, v)

        zero_rows()
        plsc.subcore_barrier()

        for dst_hbm, eh_hbm, oagg_hbm, ocnt_hbm, last in (
                (dst1_hbm, eh1_hbm, oagg1_hbm, ocnt1_hbm, False),
                (dst2_hbm, eh2_hbm, oagg2_hbm, ocnt2_hbm, True)):

            @pl.loop(0, CH)
            def _run(c, eh_hbm=eh_hbm, dst_hbm=dst_hbm):
                pltpu.sync_copy(eh_hbm.at[pl.ds(base + c * K, K)], ebufs[0])
                pltpu.sync_copy(dst_hbm.at[pl.ds(base + c * K, K)], ibufs[0])
                rewrite(0)
                pltpu.sync_copy(ebufs[0], sagg.at[lbufs[0]], add=True)

            plsc.subcore_barrier()

            wbase = sid * WROWS
            pltpu.sync_copy(sagg.at[pl.ds(wbase, WROWS)],
                            oagg_hbm.at[cid].at[pl.ds(wbase, WROWS)])
            zero_rows()
            plsc.subcore_barrier()

            fill_ones()

            @pl.loop(0, CH)
            def _cnt(c, dst_hbm=dst_hbm):
                pltpu.sync_copy(dst_hbm.at[pl.ds(base + c * K, K)], ibufs[0])
                rewrite(0)
                pltpu.sync_copy(ebufs[0], sagg.at[lbufs[0]], add=True)

            plsc.subcore_barrier()

            pltpu.sync_copy(sagg.at[pl.ds(wbase, WROWS)],
                            ocnt_hbm.at[cid].at[pl.ds(wbase, WROWS)])

            if not last:
                zero_rows()
                plsc.subcore_barrier()

    return kern(dst2d_1, eh_1, dst2d_2, eh_2)


def _node_body(x_ref, agg_ref, cnt_ref, bc_ref, br_ref, u_ref,
               wn1x_ref, wn1a_ref, wn1u_ref, bn1_ref, wn2_ref, bn2_ref,
               wn3_ref, bn3_ref, wa1_ref, ba1_ref, wa2_ref, ba2_ref,
               wa3_ref, ba3_ref, g_ref):
    i = pl.program_id(0)
    blk = x_ref.shape[0]
    cnt = cnt_ref[:, 0:1]
    agg = agg_ref[...] / jnp.maximum(cnt, 1.0)
    oh = (lax.broadcasted_iota(jnp.int32, (blk, B), 1) == bc_ref[...]
          ).astype(_bf16)
    uw = _dot(u_ref[...], wn1u_ref[...])
    t = (_dot(x_ref[...], wn1x_ref[...]) + _dot(agg, wn1a_ref[...]) +
         jnp.dot(oh, uw.astype(_bf16), preferred_element_type=_f32) +
         bn1_ref[...])
    t = jnp.maximum(t, 0.0)
    t = jnp.maximum(_dot(t, wn2_ref[...]) + bn2_ref[...], 0.0)
    xh = _dot(t, wn3_ref[...]) + bn3_ref[...]
    a = jnp.maximum(_dot(xh, wa1_ref[...]) + ba1_ref[...], 0.0)
    a = jnp.maximum(_dot(a, wa2_ref[...]) + ba2_ref[...], 0.0)
    a = _dot(a, wa3_ref[...]) + ba3_ref[...]
    c = jax.nn.sigmoid(a) * xh
    oht = (lax.broadcasted_iota(jnp.int32, (B, blk), 0) == br_ref[...]
           ).astype(_bf16)
    gp = jnp.dot(oht, c.astype(_bf16), preferred_element_type=_f32)

    @pl.when(i == 0)
    def _():
        g_ref[...] = gp

    @pl.when(i > 0)
    def _():
        g_ref[...] += gp


def _node_call(xp, agg, cnt, bc, br, u, wn1x, wn1a, wn1u, bn1, wn2, bn2,
               wn3, bn3, wa1, ba1, wa2, ba2, wa3, ba3):
    blk = 1024
    grid = NPAD // blk
    full = lambda a, b: pl.BlockSpec((a, b), lambda i: (0, 0))
    return pl.pallas_call(
        _node_body,
        grid=(grid,),
        in_specs=[
            pl.BlockSpec((blk, F_X), lambda i: (i, 0)),
            pl.BlockSpec((blk, H), lambda i: (i, 0)),
            pl.BlockSpec((blk, H), lambda i: (i, 0)),
            pl.BlockSpec((blk, 1), lambda i: (i, 0)),
            pl.BlockSpec((1, blk), lambda i: (0, i)),
            full(B, F_X),
            full(F_X, H), full(H, H), full(F_X, H), full(1, H),
            full(H, H), full(1, H), full(H, H), full(1, H),
            full(H, H), full(1, H), full(H, H), full(1, H),
            full(H, H), full(1, H),
        ],
        out_specs=pl.BlockSpec((B, H), lambda i: (0, 0)),
        out_shape=jax.ShapeDtypeStruct((B, H), _f32),
    )(xp, agg, cnt, bc, br, u, wn1x, wn1a, wn1u, bn1, wn2, bn2, wn3, bn3,
      wa1, ba1, wa2, ba2, wa3, ba3)


def _final_body(g1_ref, u1_ref, g2_ref, u2_ref,
                wg1g_ref, wg1u_ref, bg1_ref, wg2_ref, bg2_ref, wg3_ref,
                bg3_ref, wf1a_ref, wf1b_ref, bf1_ref, wf2_ref, bf2_ref,
                wf3_ref, bf3_ref, o_ref):
    def glob(g, u):
        t = jnp.maximum(_dot(g, wg1g_ref[...]) + _dot(u, wg1u_ref[...]) +
                        bg1_ref[...], 0.0)
        t = jnp.maximum(_dot(t, wg2_ref[...]) + bg2_ref[...], 0.0)
        return _dot(t, wg3_ref[...]) + bg3_ref[...]

    u1h = glob(g1_ref[...], u1_ref[...])
    u2h = glob(g2_ref[...], u2_ref[...])
    t = jnp.maximum(_dot(u1h, wf1a_ref[...]) + _dot(u2h, wf1b_ref[...]) +
                    bf1_ref[...], 0.0)
    t = jnp.maximum(_dot(t, wf2_ref[...]) + bf2_ref[...], 0.0)
    o_ref[...] = _dot(t, wf3_ref[...]) + bf3_ref[...]


def _final_call(g1, u1, g2, u2, wg1g, wg1u, bg1, wg2, bg2, wg3, bg3,
                wf1a, wf1b, bf1, wf2, bf2, wf3, bf3):
    return pl.pallas_call(
        _final_body,
        out_shape=jax.ShapeDtypeStruct((B, H), _f32),
    )(g1, u1, g2, u2, wg1g, wg1u, bg1, wg2, bg2, wg3, bg3,
      wf1a, wf1b, bf1, wf2, bf2, wf3, bf3)


def kernel(x1, edge_index1, e1, u1, batch1, x2, edge_index2, e2, u2, batch2,
           params):
    (w1, b1), (w2, b2), (w3, b3) = params["edge"]
    (wn1, bn1), (wn2, bn2), (wn3, bn3) = params["node"]
    (wa1, ba1), (wa2, ba2), (wa3, ba3) = params["attn"]
    (wg1, bg1), (wg2, bg2), (wg3, bg3) = params["glob"]
    (wf1, bf1), (wf2, bf2), (wf3, bf3) = params["final"]

    bf = lambda w: w.astype(_bf16)
    row = lambda v: v.reshape(1, -1)

    w1s, w1d = bf(w1[:F_X]), bf(w1[F_X:2 * F_X])
    w1e, w1u = bf(w1[2 * F_X:2 * F_X + F_E]), bf(w1[2 * F_X + F_E:])
    wn1x, wn1a, wn1u = bf(wn1[:F_X]), bf(wn1[F_X:2 * F_X]), bf(wn1[2 * F_X:])
    wg1g, wg1u = bf(wg1[:H]), bf(wg1[H:])
    wf1a, wf1b = bf(wf1[:H]), bf(wf1[H:])

    def prep_stage(x, edge_index, e, u, batch):
        xp = jnp.pad(x, ((0, NPAD - N), (0, 0)))
        bc = jnp.pad(batch, (0, NPAD - N), constant_values=B).reshape(NPAD, 1)
        br = bc.reshape(1, NPAD)
        src = jnp.pad(edge_index[0], (0, EPAD - E))
        dst = jnp.pad(edge_index[1], (0, EPAD - E), constant_values=N)
        ep = jnp.pad(e, ((0, EPAD - E), (0, 0)))
        pa, xb = _prep_call(xp, bc, u, w1s, w1d, w1u, row(b1))
        return xp, bc, br, src, dst, ep, pa, xb

    xp1, bc1, br1, src1, dst1, ep1, pa1, xb1 = prep_stage(
        x1, edge_index1, e1, u1, batch1)
    xp2, bc2, br2, src2, dst2, ep2, pa2, xb2 = prep_stage(
        x2, edge_index2, e2, u2, batch2)

    pa_all = jnp.concatenate([pa1, pa2])
    xb_all = jnp.concatenate([xb1, xb2])
    srcs = jnp.concatenate([src1, src2 + NPAD]).reshape(1, 2 * EPAD)
    dsts = jnp.concatenate([dst1, dst2 + NPAD]).reshape(1, 2 * EPAD)
    ga, gb = _gather_sc(pa_all, xb_all, srcs, dsts)
    goff2 = EPAD

    eh1 = _edge_call(ga, gb, ep1, w1e, bf(w2), row(b2), bf(w3), row(b3), 0)
    eh2 = _edge_call(ga, gb, ep2, w1e, bf(w2), row(b2), bf(w3), row(b3), goff2)

    aggp1, cntp1, aggp2, cntp2 = _scatter_sc(dst1, eh1, dst2, eh2)
    agg1, cnt1 = aggp1.reshape(NPAD, H), cntp1.reshape(NPAD, H)
    agg2, cnt2 = aggp2.reshape(NPAD, H), cntp2.reshape(NPAD, H)

    def node_stage(xp, bc, br, u, agg, cnt):
        return _node_call(xp, agg, cnt, bc, br, u,
                          wn1x, wn1a, wn1u, row(bn1), bf(wn2), row(bn2),
                          bf(wn3), row(bn3), bf(wa1), row(ba1), bf(wa2),
                          row(ba2), bf(wa3), row(ba3))

    g1 = node_stage(xp1, bc1, br1, u1, agg1, cnt1)
    g2 = node_stage(xp2, bc2, br2, u2, agg2, cnt2)
    return _final_call(g1, u1, g2, u2, wg1g, wg1u, row(bg1), bf(wg2),
                       row(bg2), bf(wg3), row(bg3), wf1a, wf1b, row(bf1),
                       bf(wf2), row(bf2), bf(wf3), row(bf3))


# speedup vs baseline: 3.7824x; 1.1344x over previous
import functools

import jax
import jax.numpy as jnp
from jax import lax
from jax.experimental import pallas as pl
from jax.experimental.pallas import tpu as pltpu
from jax.experimental.pallas import tpu_sc as plsc

N = 10000
E = 320000
B = 8
F_X = 128
F_E = 16
H = 128

NC = 2
NS = 16
NW = NC * NS

NPAD = 10240
EPAD = 327680
KG = 80
K = 128
RPC = NPAD // NC
SROWS = RPC + 128

_f32 = jnp.float32
_bf16 = jnp.bfloat16


@functools.cache
def _sc_mesh():
    return plsc.VectorSubcoreMesh(
        core_axis_name="c", subcore_axis_name="s", num_cores=NC,
        num_subcores=NS)


def _dot(a, w):
    return jnp.dot(a.astype(_bf16), w, preferred_element_type=_f32)


def _prep_body(x_ref, bc_ref, u_ref, w1s_ref, w1d_ref, w1u_ref, b1_ref,
               pa_ref, xb_ref):
    x = x_ref[...]
    oh = (lax.broadcasted_iota(jnp.int32, (x.shape[0], B), 1) == bc_ref[...]
          ).astype(_bf16)
    uw = _dot(u_ref[...], w1u_ref[...])
    pa_ref[...] = (_dot(x, w1s_ref[...]) +
                   jnp.dot(oh, uw.astype(_bf16), preferred_element_type=_f32) +
                   b1_ref[...])
    xb_ref[...] = _dot(x, w1d_ref[...])


def _prep_call(xp, bc, u, w1s, w1d, w1u, b1):
    blk = 1024
    grid = NPAD // blk
    return pl.pallas_call(
        _prep_body,
        grid=(grid,),
        in_specs=[
            pl.BlockSpec((blk, F_X), lambda i: (i, 0)),
            pl.BlockSpec((blk, 1), lambda i: (i, 0)),
            pl.BlockSpec((B, F_X), lambda i: (0, 0)),
            pl.BlockSpec((F_X, H), lambda i: (0, 0)),
            pl.BlockSpec((F_X, H), lambda i: (0, 0)),
            pl.BlockSpec((F_X, H), lambda i: (0, 0)),
            pl.BlockSpec((1, H), lambda i: (0, 0)),
        ],
        out_specs=[
            pl.BlockSpec((blk, H), lambda i: (i, 0)),
            pl.BlockSpec((blk, H), lambda i: (i, 0)),
        ],
        out_shape=[
            jax.ShapeDtypeStruct((NPAD, H), _f32),
            jax.ShapeDtypeStruct((NPAD, H), _f32),
        ],
    )(xp, bc, u, w1s, w1d, w1u, b1)


def _gather_sc(pa_all, xb_all, srcs, dsts):
    @functools.partial(
        pl.kernel,
        out_type=[
            jax.ShapeDtypeStruct((2 * EPAD, H), _f32),
            jax.ShapeDtypeStruct((2 * EPAD, H), _f32),
        ],
        mesh=_sc_mesh(),
        scratch_types=[pltpu.SemaphoreType.DMA, pltpu.SemaphoreType.DMA],
    )
    def kern(pa_h, xb_h, src_h, dst_h, ga_h, gb_h, sem_a, sem_b):
        def body(ia, ib, oa, ob):
            da = pltpu.async_copy(pa_h.at[ia.at[0]], oa, sem_a)
            db = pltpu.async_copy(xb_h.at[ib.at[0]], ob, sem_b)
            da.wait()
            db.wait()

        pltpu.emit_pipeline(
            body,
            grid=(2 * EPAD // K,),
            in_specs=[
                pl.BlockSpec((1, K), lambda i: (0, i)),
                pl.BlockSpec((1, K), lambda i: (0, i)),
            ],
            out_specs=[
                pl.BlockSpec((K, H), lambda i: (i, 0)),
                pl.BlockSpec((K, H), lambda i: (i, 0)),
            ],
            core_axis_name=("c", "s"),
            dimension_semantics=(pltpu.PARALLEL,),
        )(src_h, dst_h, ga_h, gb_h)

    return kern(pa_all, xb_all, srcs, dsts)


def _edge_body(ga_ref, gb_ref, e_ref, w1e_ref, w2_ref, b2_ref, w3_ref, b3_ref,
               eh_ref):
    h1 = ga_ref[...] + gb_ref[...] + _dot(e_ref[...], w1e_ref[...])
    h1 = jnp.maximum(h1, 0.0)
    h2 = jnp.maximum(_dot(h1, w2_ref[...]) + b2_ref[...], 0.0)
    eh_ref[...] = _dot(h2, w3_ref[...]) + b3_ref[...]


def _edge_call(ga, gb, ep, w1e, w2, b2, w3, b3, goff):
    blk = 2048
    grid = EPAD // blk
    ob = goff // blk
    return pl.pallas_call(
        _edge_body,
        grid=(grid,),
        in_specs=[
            pl.BlockSpec((blk, H), lambda i: (i + ob, 0)),
            pl.BlockSpec((blk, H), lambda i: (i + ob, 0)),
            pl.BlockSpec((blk, F_E), lambda i: (i, 0)),
            pl.BlockSpec((F_E, H), lambda i: (0, 0)),
            pl.BlockSpec((H, H), lambda i: (0, 0)),
            pl.BlockSpec((1, H), lambda i: (0, 0)),
            pl.BlockSpec((H, H), lambda i: (0, 0)),
            pl.BlockSpec((1, H), lambda i: (0, 0)),
        ],
        out_specs=pl.BlockSpec((blk, H), lambda i: (i, 0)),
        out_shape=jax.ShapeDtypeStruct((EPAD, H), _f32),
    )(ga, gb, ep, w1e, w2, b2, w3, b3)


def _scatter_sc(dst2d_1, eh_1, dst2d_2, eh_2):
    SEPT = EPAD // NS
    CH = SEPT // K
    ZB = SROWS // NS // 8
    WROWS = RPC // NS

    @functools.partial(
        pl.kernel,
        out_type=[
            jax.ShapeDtypeStruct((NC, RPC, H), _f32),
            jax.ShapeDtypeStruct((NC, RPC, H), _f32),
            jax.ShapeDtypeStruct((NC, RPC, H), _f32),
            jax.ShapeDtypeStruct((NC, RPC, H), _f32),
        ],
        mesh=_sc_mesh(),
        scratch_types=[
            pltpu.VMEM((8, H), _f32),
            pltpu.VMEM((K, H), _f32),
            pltpu.VMEM((K,), jnp.int32),
            pltpu.VMEM((K,), jnp.int32),
            pltpu.VMEM_SHARED((SROWS, H), _f32),
        ],
    )
    def kern(dst1_hbm, eh1_hbm, dst2_hbm, eh2_hbm,
             oagg1_hbm, ocnt1_hbm, oagg2_hbm, ocnt2_hbm,
             zrow_v, eb0, ib0, lb0, sagg):
        cid = lax.axis_index("c")
        sid = lax.axis_index("s")
        base = sid * SEPT

        @pl.loop(0, 8)
        def _fill(r):
            @pl.loop(0, H // 16)
            def _z(cc):
                zrow_v[r, pl.ds(cc * 16, 16)] = jnp.zeros((16,), _f32)

        def fill_ones():
            @pl.loop(0, K)
            def _o1(r):
                @pl.loop(0, H // 16)
                def _o2(cc):
                    eb0[r, pl.ds(cc * 16, 16)] = jnp.full((16,), 1.0, _f32)

        zbase = sid * (SROWS // NS)

        def zero_rows():
            @pl.loop(0, ZB)
            def _zero(zi):
                pltpu.sync_copy(zrow_v, sagg.at[pl.ds(zbase + zi * 8, 8)])

        ebufs = (eb0,)
        ibufs = (ib0,)
        lbufs = (lb0,)

        def rewrite(b):
            @pl.loop(0, K // 16)
            def _rw(j):
                v = ibufs[b][pl.ds(j * 16, 16)]
                v = v - cid * RPC
                oob = (v < 0) | (v >= RPC)
                lbufs[b][pl.ds(j * 16, 16)] = jnp.where(oob, RPC, v)

        zero_rows()
        plsc.subcore_barrier()

        for dst_hbm, eh_hbm, oagg_hbm, ocnt_hbm, last in (
                (dst1_hbm, eh1_hbm, oagg1_hbm, ocnt1_hbm, False),
                (dst2_hbm, eh2_hbm, oagg2_hbm, ocnt2_hbm, True)):

            @pl.loop(0, CH)
            def _run(c, eh_hbm=eh_hbm, dst_hbm=dst_hbm):
                pltpu.sync_copy(eh_hbm.at[pl.ds(base + c * K, K)], ebufs[0])
                pltpu.sync_copy(dst_hbm.at[pl.ds(base + c * K, K)], ibufs[0])
                rewrite(0)
                pltpu.sync_copy(ebufs[0], sagg.at[lbufs[0]], add=True)

            plsc.subcore_barrier()

            wbase = sid * WROWS
            pltpu.sync_copy(sagg.at[pl.ds(wbase, WROWS)],
                            oagg_hbm.at[cid].at[pl.ds(wbase, WROWS)])
            zero_rows()
            plsc.subcore_barrier()

            fill_ones()

            @pl.loop(0, CH)
            def _cnt(c, dst_hbm=dst_hbm):
                pltpu.sync_copy(dst_hbm.at[pl.ds(base + c * K, K)], ibufs[0])
                rewrite(0)
                pltpu.sync_copy(ebufs[0], sagg.at[lbufs[0]], add=True)

            plsc.subcore_barrier()

            pltpu.sync_copy(sagg.at[pl.ds(wbase, WROWS)],
                            ocnt_hbm.at[cid].at[pl.ds(wbase, WROWS)])

            if not last:
                zero_rows()
                plsc.subcore_barrier()

    return kern(dst2d_1, eh_1, dst2d_2, eh_2)


def _node_body(x_ref, agg_ref, cnt_ref, bc_ref, br_ref, u_ref,
               wn1x_ref, wn1a_ref, wn1u_ref, bn1_ref, wn2_ref, bn2_ref,
               wn3_ref, bn3_ref, wa1_ref, ba1_ref, wa2_ref, ba2_ref,
               wa3_ref, ba3_ref, g_ref):
    i = pl.program_id(0)
    blk = x_ref.shape[0]
    cnt = cnt_ref[:, 0:1]
    agg = agg_ref[...] / jnp.maximum(cnt, 1.0)
    oh = (lax.broadcasted_iota(jnp.int32, (blk, B), 1) == bc_ref[...]
          ).astype(_bf16)
    uw = _dot(u_ref[...], wn1u_ref[...])
    t = (_dot(x_ref[...], wn1x_ref[...]) + _dot(agg, wn1a_ref[...]) +
         jnp.dot(oh, uw.astype(_bf16), preferred_element_type=_f32) +
         bn1_ref[...])
    t = jnp.maximum(t, 0.0)
    t = jnp.maximum(_dot(t, wn2_ref[...]) + bn2_ref[...], 0.0)
    xh = _dot(t, wn3_ref[...]) + bn3_ref[...]
    a = jnp.maximum(_dot(xh, wa1_ref[...]) + ba1_ref[...], 0.0)
    a = jnp.maximum(_dot(a, wa2_ref[...]) + ba2_ref[...], 0.0)
    a = _dot(a, wa3_ref[...]) + ba3_ref[...]
    c = jax.nn.sigmoid(a) * xh
    oht = (lax.broadcasted_iota(jnp.int32, (B, blk), 0) == br_ref[...]
           ).astype(_bf16)
    gp = jnp.dot(oht, c.astype(_bf16), preferred_element_type=_f32)

    @pl.when(i == 0)
    def _():
        g_ref[...] = gp

    @pl.when(i > 0)
    def _():
        g_ref[...] += gp


def _node_call(xp, agg, cnt, bc, br, u, wn1x, wn1a, wn1u, bn1, wn2, bn2,
               wn3, bn3, wa1, ba1, wa2, ba2, wa3, ba3):
    blk = 1024
    grid = NPAD // blk
    full = lambda a, b: pl.BlockSpec((a, b), lambda i: (0, 0))
    return pl.pallas_call(
        _node_body,
        grid=(grid,),
        in_specs=[
            pl.BlockSpec((blk, F_X), lambda i: (i, 0)),
            pl.BlockSpec((blk, H), lambda i: (i, 0)),
            pl.BlockSpec((blk, H), lambda i: (i, 0)),
            pl.BlockSpec((blk, 1), lambda i: (i, 0)),
            pl.BlockSpec((1, blk), lambda i: (0, i)),
            full(B, F_X),
            full(F_X, H), full(H, H), full(F_X, H), full(1, H),
            full(H, H), full(1, H), full(H, H), full(1, H),
            full(H, H), full(1, H), full(H, H), full(1, H),
            full(H, H), full(1, H),
        ],
        out_specs=pl.BlockSpec((B, H), lambda i: (0, 0)),
        out_shape=jax.ShapeDtypeStruct((B, H), _f32),
    )(xp, agg, cnt, bc, br, u, wn1x, wn1a, wn1u, bn1, wn2, bn2, wn3, bn3,
      wa1, ba1, wa2, ba2, wa3, ba3)


def _final_body(g1_ref, u1_ref, g2_ref, u2_ref,
                wg1g_ref, wg1u_ref, bg1_ref, wg2_ref, bg2_ref, wg3_ref,
                bg3_ref, wf1a_ref, wf1b_ref, bf1_ref, wf2_ref, bf2_ref,
                wf3_ref, bf3_ref, o_ref):
    def glob(g, u):
        t = jnp.maximum(_dot(g, wg1g_ref[...]) + _dot(u, wg1u_ref[...]) +
                        bg1_ref[...], 0.0)
        t = jnp.maximum(_dot(t, wg2_ref[...]) + bg2_ref[...], 0.0)
        return _dot(t, wg3_ref[...]) + bg3_ref[...]

    u1h = glob(g1_ref[...], u1_ref[...])
    u2h = glob(g2_ref[...], u2_ref[...])
    t = jnp.maximum(_dot(u1h, wf1a_ref[...]) + _dot(u2h, wf1b_ref[...]) +
                    bf1_ref[...], 0.0)
    t = jnp.maximum(_dot(t, wf2_ref[...]) + bf2_ref[...], 0.0)
    o_ref[...] = _dot(t, wf3_ref[...]) + bf3_ref[...]


def _final_call(g1, u1, g2, u2, wg1g, wg1u, bg1, wg2, bg2, wg3, bg3,
                wf1a, wf1b, bf1, wf2, bf2, wf3, bf3):
    return pl.pallas_call(
        _final_body,
        out_shape=jax.ShapeDtypeStruct((B, H), _f32),
    )(g1, u1, g2, u2, wg1g, wg1u, bg1, wg2, bg2, wg3, bg3,
      wf1a, wf1b, bf1, wf2, bf2, wf3, bf3)


def kernel(x1, edge_index1, e1, u1, batch1, x2, edge_index2, e2, u2, batch2,
           params):
    (w1, b1), (w2, b2), (w3, b3) = params["edge"]
    (wn1, bn1), (wn2, bn2), (wn3, bn3) = params["node"]
    (wa1, ba1), (wa2, ba2), (wa3, ba3) = params["attn"]
    (wg1, bg1), (wg2, bg2), (wg3, bg3) = params["glob"]
    (wf1, bf1), (wf2, bf2), (wf3, bf3) = params["final"]

    bf = lambda w: w.astype(_bf16)
    row = lambda v: v.reshape(1, -1)

    w1s, w1d = bf(w1[:F_X]), bf(w1[F_X:2 * F_X])
    w1e, w1u = bf(w1[2 * F_X:2 * F_X + F_E]), bf(w1[2 * F_X + F_E:])
    wn1x, wn1a, wn1u = bf(wn1[:F_X]), bf(wn1[F_X:2 * F_X]), bf(wn1[2 * F_X:])
    wg1g, wg1u = bf(wg1[:H]), bf(wg1[H:])
    wf1a, wf1b = bf(wf1[:H]), bf(wf1[H:])

    def prep_stage(x, edge_index, e, u, batch):
        xp = jnp.pad(x, ((0, NPAD - N), (0, 0)))
        bc = jnp.pad(batch, (0, NPAD - N), constant_values=B).reshape(NPAD, 1)
        br = bc.reshape(1, NPAD)
        src = jnp.pad(edge_index[0], (0, EPAD - E))
        dst = jnp.pad(edge_index[1], (0, EPAD - E), constant_values=N)
        ep = jnp.pad(e, ((0, EPAD - E), (0, 0)))
        pa, xb = _prep_call(xp, bc, u, w1s, w1d, w1u, row(b1))
        return xp, bc, br, src, dst, ep, pa, xb

    xp1, bc1, br1, src1, dst1, ep1, pa1, xb1 = prep_stage(
        x1, edge_index1, e1, u1, batch1)
    xp2, bc2, br2, src2, dst2, ep2, pa2, xb2 = prep_stage(
        x2, edge_index2, e2, u2, batch2)

    pa_all = jnp.concatenate([pa1, pa2])
    xb_all = jnp.concatenate([xb1, xb2])
    srcs = jnp.concatenate([src1, src2 + NPAD]).reshape(1, 2 * EPAD)
    dsts = jnp.concatenate([dst1, dst2 + NPAD]).reshape(1, 2 * EPAD)
    ga, gb = _gather_sc(pa_all, xb_all, srcs, dsts)
    goff2 = EPAD

    eh1 = _edge_call(ga, gb, ep1, w1e, bf(w2), row(b2), bf(w3), row(b3), 0)
    eh2 = _edge_call(ga, gb, ep2, w1e, bf(w2), row(b2), bf(w3), row(b3), goff2)

    aggp1, cntp1, aggp2, cntp2 = _scatter_sc(dst1, eh1, dst2, eh2)
    agg1, cnt1 = aggp1.reshape(NPAD, H), cntp1.reshape(NPAD, H)
    agg2, cnt2 = aggp2.reshape(NPAD, H), cntp2.reshape(NPAD, H)

    def node_stage(xp, bc, br, u, agg, cnt):
        return _node_call(xp, agg, cnt, bc, br, u,
                          wn1x, wn1a, wn1u, row(bn1), bf(wn2), row(bn2),
                          bf(wn3), row(bn3), bf(wa1), row(ba1), bf(wa2),
                          row(ba2), bf(wa3), row(ba3))

    g1 = node_stage(xp1, bc1, br1, u1, agg1, cnt1)
    g2 = node_stage(xp2, bc2, br2, u2, agg2, cnt2)
    return _final_call(g1, u1, g2, u2, wg1g, wg1u, row(bg1), bf(wg2),
                       row(bg2), bf(wg3), row(bg3), wf1a, wf1b, row(bf1),
                       bf(wf2), row(bf2), bf(wf3), row(bf3))
